# baseline (speedup 1.0000x reference)
import jax
import jax.numpy as jnp
from jax import lax
from jax.experimental import pallas as pl
from jax.experimental.pallas import tpu as pltpu
from jax.experimental.pallas import tpu_sc as plsc

N_NODES = 10000
N_EDGES = 320000
HID = 128
NLAYERS = 3
NRBF = 16
NV = 21

NCORES = 2
NSUB = 16
NTILES = NCORES * NSUB
EDGES_PER_TILE = N_EDGES // NTILES
CHUNK = 16
NCHUNKS = EDGES_PER_TILE // CHUNK
NBUF = 3
PACK_SHIFT = 14
N_PAD = 10240
ROW_A = N_PAD // NSUB

EB = 512
NB = 2000

_f32 = jnp.float32


def _edge_feat_body(d_ref, We_ref, be_ref, W3_ref, bm_ref, o0_ref, o1_ref, o2_ref):
    d = d_ref[:]
    centers = (lax.broadcasted_iota(jnp.int32, (1, NRBF), 1).astype(_f32)
               * (20.0 / (NRBF - 1)))
    z = (d[:, None] - centers) * (NRBF / 20.0)
    rbf = jnp.exp(-(z * z))
    e = jnp.maximum(jnp.dot(rbf, We_ref[:], preferred_element_type=_f32)
                    + be_ref[:], 0.0)
    o0_ref[:] = jnp.dot(e, W3_ref[0], preferred_element_type=_f32) + bm_ref[0][None, :]
    o1_ref[:] = jnp.dot(e, W3_ref[1], preferred_element_type=_f32) + bm_ref[1][None, :]
    o2_ref[:] = jnp.dot(e, W3_ref[2], preferred_element_type=_f32) + bm_ref[2][None, :]


def _edge_feat(dists, W_edge, be2d, W3, b_msg):
    grid = (N_EDGES // EB,)
    return pl.pallas_call(
        _edge_feat_body,
        grid=grid,
        in_specs=[
            pl.BlockSpec((EB,), lambda i: (i,)),
            pl.BlockSpec((NRBF, HID), lambda i: (0, 0)),
            pl.BlockSpec((1, HID), lambda i: (0, 0)),
            pl.BlockSpec((NLAYERS, HID, HID), lambda i: (0, 0, 0)),
            pl.BlockSpec((NLAYERS, HID), lambda i: (0, 0)),
        ],
        out_specs=[pl.BlockSpec((EB, HID), lambda i: (i, 0))] * 3,
        out_shape=[jax.ShapeDtypeStruct((N_EDGES, HID), _f32)] * 3,
    )(dists, W_edge, be2d, W3, b_msg)


def _node_embed_body(cp_ref, Wn_ref, bn_ref, W1_ref, W2_ref, h_ref, a_ref, b_ref):
    h = jnp.maximum(jnp.dot(cp_ref[:], Wn_ref[:], preferred_element_type=_f32)
                    + bn_ref[:], 0.0)
    h_ref[:] = h
    a_ref[:] = jnp.dot(h, W1_ref[:], preferred_element_type=_f32)
    b_ref[:] = jnp.dot(h, W2_ref[:], preferred_element_type=_f32)


def _node_embed(coords_p, Wn_p, bn2d, W1_0, W2_0):
    grid = (N_NODES // NB,)
    return pl.pallas_call(
        _node_embed_body,
        grid=grid,
        in_specs=[
            pl.BlockSpec((NB, HID), lambda i: (i, 0)),
            pl.BlockSpec((HID, HID), lambda i: (0, 0)),
            pl.BlockSpec((1, HID), lambda i: (0, 0)),
            pl.BlockSpec((HID, HID), lambda i: (0, 0)),
            pl.BlockSpec((HID, HID), lambda i: (0, 0)),
        ],
        out_specs=[pl.BlockSpec((NB, HID), lambda i: (i, 0))] * 3,
        out_shape=[jax.ShapeDtypeStruct((N_NODES, HID), _f32)] * 3,
    )(coords_p, Wn_p, bn2d, W1_0, W2_0)


def _ln_update(p_ref, h_ref, Wu_ref, bu_ref, lns_ref, lnb_ref):
    agg = p_ref[0] + p_ref[1]
    upd = jnp.maximum(jnp.dot(agg, Wu_ref[:], preferred_element_type=_f32)
                      + bu_ref[:], 0.0)
    x = h_ref[:] + upd
    mu = jnp.mean(x, axis=1, keepdims=True)
    xc = x - mu
    var = jnp.mean(xc * xc, axis=1, keepdims=True)
    return xc / jnp.sqrt(var + 1e-5) * lns_ref[:] + lnb_ref[:]


def _update_ab_body(p_ref, h_ref, Wu_ref, bu_ref, lns_ref, lnb_ref,
                    W1_ref, W2_ref, h_out, a_out, b_out):
    hn = _ln_update(p_ref, h_ref, Wu_ref, bu_ref, lns_ref, lnb_ref)
    h_out[:] = hn
    a_out[:] = jnp.dot(hn, W1_ref[:], preferred_element_type=_f32)
    b_out[:] = jnp.dot(hn, W2_ref[:], preferred_element_type=_f32)


def _update_logits_body(p_ref, h_ref, Wu_ref, bu_ref, lns_ref, lnb_ref,
                        Wo_ref, bo_ref, o_ref):
    hn = _ln_update(p_ref, h_ref, Wu_ref, bu_ref, lns_ref, lnb_ref)
    o_ref[:] = (jnp.dot(hn, Wo_ref[:], preferred_element_type=_f32)
                + bo_ref[:]) / 0.1


_SPEC_P = pl.BlockSpec((NCORES, NB, HID), lambda i: (0, i, 0))
_SPEC_N = pl.BlockSpec((NB, HID), lambda i: (i, 0))
_SPEC_W = pl.BlockSpec((HID, HID), lambda i: (0, 0))
_SPEC_B = pl.BlockSpec((1, HID), lambda i: (0, 0))


def _update_ab(parts, h, Wu, bu2d, lns2d, lnb2d, W1n, W2n):
    grid = (N_NODES // NB,)
    return pl.pallas_call(
        _update_ab_body,
        grid=grid,
        in_specs=[_SPEC_P, _SPEC_N, _SPEC_W, _SPEC_B, _SPEC_B, _SPEC_B,
                  _SPEC_W, _SPEC_W],
        out_specs=[_SPEC_N] * 3,
        out_shape=[jax.ShapeDtypeStruct((N_NODES, HID), _f32)] * 3,
    )(parts, h, Wu, bu2d, lns2d, lnb2d, W1n, W2n)


def _update_logits(parts, h, Wu, bu2d, lns2d, lnb2d, Wo_p, bo_p):
    grid = (N_NODES // NB,)
    return pl.pallas_call(
        _update_logits_body,
        grid=grid,
        in_specs=[_SPEC_P, _SPEC_N, _SPEC_W, _SPEC_B, _SPEC_B, _SPEC_B,
                  _SPEC_W, _SPEC_B],
        out_specs=_SPEC_N,
        out_shape=jax.ShapeDtypeStruct((N_NODES, HID), _f32),
    )(parts, h, Wu, bu2d, lns2d, lnb2d, Wo_p, bo_p)


def _sc_layer_body(a_hbm, b_hbm, ew_hbm, packed_hbm, zeros_hbm, out_hbm,
                   packed_v, aidx, didx, abuf, bbuf, ebuf, acc, sems):
    cid = lax.axis_index("c")
    sid = lax.axis_index("s")
    wid = cid * NSUB + sid
    row0 = sid * ROW_A
    ebase = wid * EDGES_PER_TILE

    pltpu.sync_copy(packed_hbm.at[pl.ds(ebase, EDGES_PER_TILE)], packed_v)
    pltpu.sync_copy(zeros_hbm.at[pl.ds(row0, ROW_A)],
                    acc.at[pl.ds(row0, ROW_A)])

    plsc.subcore_barrier()

    def unpack(c, s):
        p = packed_v[pl.ds(c * CHUNK, CHUNK)]
        aidx[s][pl.ds(0, CHUNK)] = p & ((1 << PACK_SHIFT) - 1)
        didx[s][pl.ds(0, CHUNK)] = lax.shift_right_logical(p, PACK_SHIFT)

    def issue_g(c, s):
        pltpu.async_copy(a_hbm.at[aidx[s]], abuf[s], sems[s][0])
        pltpu.async_copy(b_hbm.at[didx[s]], bbuf[s], sems[s][1])
        pltpu.async_copy(ew_hbm.at[pl.ds(ebase + c * CHUNK, CHUNK)],
                         ebuf[s], sems[s][2])

    def wait_g(c, s):
        pltpu.make_async_copy(a_hbm.at[aidx[s]], abuf[s], sems[s][0]).wait()
        pltpu.make_async_copy(b_hbm.at[didx[s]], bbuf[s], sems[s][1]).wait()
        pltpu.make_async_copy(ew_hbm.at[pl.ds(ebase + c * CHUNK, CHUNK)],
                              ebuf[s], sems[s][2]).wait()

    def issue_sc(s):
        pltpu.async_copy(ebuf[s], acc.at[didx[s]], sems[s][3], add=True)

    def wait_sc(s):
        pltpu.make_async_copy(ebuf[s], acc.at[didx[s]], sems[s][3]).wait()

    def compute(s):
        a_v, b_v, e_v = abuf[s], bbuf[s], ebuf[s]

        @plsc.parallel_loop(0, CHUNK, 1, unroll=8)
        def _row(r):
            for j in range(HID // 16):
                sl = pl.ds(j * 16, 16)
                v = a_v[r, sl] + b_v[r, sl] + e_v[r, sl]
                e_v[r, sl] = jnp.maximum(v, 0.0)

    def process(c, s, issue_next):
        wait_g(c, s)
        compute(s)
        issue_sc(s)
        if issue_next:
            s2 = (s + 2) % NBUF
            wait_sc(s2)
            unpack(c + 2, s2)
            issue_g(c + 2, s2)

    unpack(0, 0)
    issue_g(0, 0)
    unpack(1, 1)
    issue_g(1, 1)
    wait_g(0, 0)
    compute(0)
    issue_sc(0)
    unpack(2, 2)
    issue_g(2, 2)
    process(1, 1, True)
    process(2, 2, True)

    def triple(t, carry):
        c0 = t * NBUF
        for k in range(NBUF):
            process(c0 + k, k, True)
        return carry

    lax.fori_loop(1, (NCHUNKS - 4) // NBUF, triple, 0)

    process(jnp.int32(NCHUNKS - 4), 0, True)
    process(jnp.int32(NCHUNKS - 3), 1, True)
    process(jnp.int32(NCHUNKS - 2), 2, False)
    process(jnp.int32(NCHUNKS - 1), 0, False)
    wait_sc(1)
    wait_sc(2)
    wait_sc(0)

    plsc.subcore_barrier()
    pltpu.sync_copy(acc.at[pl.ds(row0, ROW_A)],
                    out_hbm.at[cid, pl.ds(row0, ROW_A)])


def _sc_body_wrapper(a_hbm, b_hbm, ew_hbm, packed_hbm, zeros_hbm, out_hbm,
                     packed_v,
                     ai0, di0, ai1, di1, ai2, di2,
                     a0, b0, e0, a1, b1, e1, a2, b2, e2,
                     sga0, sgb0, sge0, ssc0,
                     sga1, sgb1, sge1, ssc1,
                     sga2, sgb2, sge2, ssc2,
                     acc):
    _sc_layer_body(a_hbm, b_hbm, ew_hbm, packed_hbm, zeros_hbm, out_hbm,
                   packed_v, (ai0, ai1, ai2), (di0, di1, di2),
                   (a0, a1, a2), (b0, b1, b2), (e0, e1, e2), acc,
                   ((sga0, sgb0, sge0, ssc0),
                    (sga1, sgb1, sge1, ssc1),
                    (sga2, sgb2, sge2, ssc2)))


_sc_layer = pl.kernel(
    _sc_body_wrapper,
    out_type=jax.ShapeDtypeStruct((NCORES, N_PAD, HID), _f32),
    mesh=plsc.VectorSubcoreMesh(core_axis_name="c", subcore_axis_name="s"),
    scratch_types=(
        [pltpu.VMEM((EDGES_PER_TILE,), jnp.int32)]
        + [pltpu.VMEM((CHUNK,), jnp.int32)] * 6
        + [pltpu.VMEM((CHUNK, HID), _f32)] * 9
        + [pltpu.SemaphoreType.DMA] * 12
        + [pltpu.VMEM_SHARED((N_PAD, HID), _f32)]
    ),
)


def kernel(node_coords, edge_index, edge_distances, W_node, b_node, W_edge,
           b_edge, W_msg, b_msg, W_upd, b_upd, ln_scale, ln_bias, W_out, b_out):
    src = edge_index[0].astype(jnp.int32)
    dst = edge_index[1].astype(jnp.int32)
    packed = src | (dst << PACK_SHIFT)
    dists = edge_distances.astype(_f32)

    coords_p = jnp.zeros((N_NODES, HID), _f32).at[:, :3].set(node_coords)
    Wn_p = jnp.zeros((HID, HID), _f32).at[:3].set(W_node)
    W1 = W_msg[:, :HID, :]
    W2 = W_msg[:, HID:2 * HID, :]
    W3 = W_msg[:, 2 * HID:, :]
    Wo_p = jnp.zeros((HID, HID), _f32).at[:, :NV].set(W_out)
    bo_p = jnp.zeros((1, HID), _f32).at[0, :NV].set(b_out)

    eW = _edge_feat(dists, W_edge, b_edge.reshape(1, HID), W3, b_msg)
    h, A, B = _node_embed(coords_p, Wn_p, b_node.reshape(1, HID), W1[0], W2[0])
    zeros_nh = jnp.zeros((N_PAD, HID), _f32)

    logits = None
    for l in range(NLAYERS):
        parts = _sc_layer(A, B, eW[l], packed, zeros_nh)
        if l + 1 < NLAYERS:
            h, A, B = _update_ab(parts, h, W_upd[l], b_upd[l].reshape(1, HID),
                                 ln_scale[l].reshape(1, HID),
                                 ln_bias[l].reshape(1, HID), W1[l + 1], W2[l + 1])
        else:
            logits = _update_logits(parts, h, W_upd[l],
                                    b_upd[l].reshape(1, HID),
                                    ln_scale[l].reshape(1, HID),
                                    ln_bias[l].reshape(1, HID), Wo_p, bo_p)
    return logits[:, :NV]


# speedup vs baseline: 1.0541x; 1.0541x over previous
import jax
import jax.numpy as jnp
from jax import lax
from jax.experimental import pallas as pl
from jax.experimental.pallas import tpu as pltpu
from jax.experimental.pallas import tpu_sc as plsc

N_NODES = 10000
N_EDGES = 320000
HID = 128
NLAYERS = 3
NRBF = 16
NV = 21

NCORES = 2
NSUB = 16
NTILES = NCORES * NSUB
EDGES_PER_TILE = N_EDGES // NTILES
CHUNK = 16
NCHUNKS = EDGES_PER_TILE // CHUNK
NBUF = 3
PACK_SHIFT = 14
N_PAD = 10240
ROW_A = N_PAD // NSUB

EB = 512
NB = 2000

_f32 = jnp.float32


def _edge_feat_body(d_ref, We_ref, be_ref, W3_ref, bm_ref, o0_ref, o1_ref, o2_ref):
    d = d_ref[:]
    centers = (lax.broadcasted_iota(jnp.int32, (1, NRBF), 1).astype(_f32)
               * (20.0 / (NRBF - 1)))
    z = (d[:, None] - centers) * (NRBF / 20.0)
    rbf = jnp.exp(-(z * z))
    e = jnp.maximum(jnp.dot(rbf, We_ref[:], preferred_element_type=_f32)
                    + be_ref[:], 0.0)
    o0_ref[:] = jnp.dot(e, W3_ref[0], preferred_element_type=_f32) + bm_ref[0][None, :]
    o1_ref[:] = jnp.dot(e, W3_ref[1], preferred_element_type=_f32) + bm_ref[1][None, :]
    o2_ref[:] = jnp.dot(e, W3_ref[2], preferred_element_type=_f32) + bm_ref[2][None, :]


def _edge_feat(dists, W_edge, be2d, W3, b_msg):
    grid = (N_EDGES // EB,)
    return pl.pallas_call(
        _edge_feat_body,
        grid=grid,
        in_specs=[
            pl.BlockSpec((EB,), lambda i: (i,)),
            pl.BlockSpec((NRBF, HID), lambda i: (0, 0)),
            pl.BlockSpec((1, HID), lambda i: (0, 0)),
            pl.BlockSpec((NLAYERS, HID, HID), lambda i: (0, 0, 0)),
            pl.BlockSpec((NLAYERS, HID), lambda i: (0, 0)),
        ],
        out_specs=[pl.BlockSpec((EB, HID), lambda i: (i, 0))] * 3,
        out_shape=[jax.ShapeDtypeStruct((N_EDGES, HID), _f32)] * 3,
    )(dists, W_edge, be2d, W3, b_msg)


def _node_embed_body(cp_ref, Wn_ref, bn_ref, W1_ref, W2_ref, h_ref, a_ref, b_ref):
    h = jnp.maximum(jnp.dot(cp_ref[:], Wn_ref[:], preferred_element_type=_f32)
                    + bn_ref[:], 0.0)
    h_ref[:] = h
    a_ref[:] = jnp.dot(h, W1_ref[:], preferred_element_type=_f32)
    b_ref[:] = jnp.dot(h, W2_ref[:], preferred_element_type=_f32)


def _node_embed(coords_p, Wn_p, bn2d, W1_0, W2_0):
    grid = (N_NODES // NB,)
    return pl.pallas_call(
        _node_embed_body,
        grid=grid,
        in_specs=[
            pl.BlockSpec((NB, HID), lambda i: (i, 0)),
            pl.BlockSpec((HID, HID), lambda i: (0, 0)),
            pl.BlockSpec((1, HID), lambda i: (0, 0)),
            pl.BlockSpec((HID, HID), lambda i: (0, 0)),
            pl.BlockSpec((HID, HID), lambda i: (0, 0)),
        ],
        out_specs=[pl.BlockSpec((NB, HID), lambda i: (i, 0))] * 3,
        out_shape=[jax.ShapeDtypeStruct((N_NODES, HID), _f32)] * 3,
    )(coords_p, Wn_p, bn2d, W1_0, W2_0)


def _ln_update(p_ref, h_ref, Wu_ref, bu_ref, lns_ref, lnb_ref):
    agg = p_ref[0] + p_ref[1]
    upd = jnp.maximum(jnp.dot(agg, Wu_ref[:], preferred_element_type=_f32)
                      + bu_ref[:], 0.0)
    x = h_ref[:] + upd
    mu = jnp.mean(x, axis=1, keepdims=True)
    xc = x - mu
    var = jnp.mean(xc * xc, axis=1, keepdims=True)
    return xc / jnp.sqrt(var + 1e-5) * lns_ref[:] + lnb_ref[:]


def _update_ab_body(p_ref, h_ref, Wu_ref, bu_ref, lns_ref, lnb_ref,
                    W1_ref, W2_ref, h_out, a_out, b_out):
    hn = _ln_update(p_ref, h_ref, Wu_ref, bu_ref, lns_ref, lnb_ref)
    h_out[:] = hn
    a_out[:] = jnp.dot(hn, W1_ref[:], preferred_element_type=_f32)
    b_out[:] = jnp.dot(hn, W2_ref[:], preferred_element_type=_f32)


def _update_logits_body(p_ref, h_ref, Wu_ref, bu_ref, lns_ref, lnb_ref,
                        Wo_ref, bo_ref, o_ref):
    hn = _ln_update(p_ref, h_ref, Wu_ref, bu_ref, lns_ref, lnb_ref)
    o_ref[:] = (jnp.dot(hn, Wo_ref[:], preferred_element_type=_f32)
                + bo_ref[:]) / 0.1


_SPEC_P = pl.BlockSpec((NCORES, NB, HID), lambda i: (0, i, 0))
_SPEC_N = pl.BlockSpec((NB, HID), lambda i: (i, 0))
_SPEC_W = pl.BlockSpec((HID, HID), lambda i: (0, 0))
_SPEC_B = pl.BlockSpec((1, HID), lambda i: (0, 0))


def _update_ab(parts, h, Wu, bu2d, lns2d, lnb2d, W1n, W2n):
    grid = (N_NODES // NB,)
    return pl.pallas_call(
        _update_ab_body,
        grid=grid,
        in_specs=[_SPEC_P, _SPEC_N, _SPEC_W, _SPEC_B, _SPEC_B, _SPEC_B,
                  _SPEC_W, _SPEC_W],
        out_specs=[_SPEC_N] * 3,
        out_shape=[jax.ShapeDtypeStruct((N_NODES, HID), _f32)] * 3,
    )(parts, h, Wu, bu2d, lns2d, lnb2d, W1n, W2n)


def _update_logits(parts, h, Wu, bu2d, lns2d, lnb2d, Wo_p, bo_p):
    grid = (N_NODES // NB,)
    return pl.pallas_call(
        _update_logits_body,
        grid=grid,
        in_specs=[_SPEC_P, _SPEC_N, _SPEC_W, _SPEC_B, _SPEC_B, _SPEC_B,
                  _SPEC_W, _SPEC_B],
        out_specs=_SPEC_N,
        out_shape=jax.ShapeDtypeStruct((N_NODES, HID), _f32),
    )(parts, h, Wu, bu2d, lns2d, lnb2d, Wo_p, bo_p)


def _sc_layer_body(a_hbm, b_hbm, ew_hbm, packed_hbm, zeros_hbm, out_hbm,
                   packed_v, aidx, didx, sidx, abuf, bbuf, ebuf, acc, sems):
    cid = lax.axis_index("c")
    sid = lax.axis_index("s")
    wid = cid * NSUB + sid
    row0 = sid * ROW_A
    ebase = wid * EDGES_PER_TILE

    pltpu.sync_copy(packed_hbm.at[pl.ds(ebase, EDGES_PER_TILE)], packed_v)
    pltpu.sync_copy(zeros_hbm.at[pl.ds(row0, ROW_A)],
                    acc.at[pl.ds(row0, ROW_A)])

    plsc.subcore_barrier()

    def unpack(c, s):
        p = packed_v[pl.ds(c * CHUNK, CHUNK)]
        aidx[s][pl.ds(0, CHUNK)] = p & ((1 << PACK_SHIFT) - 1)
        didx[s][pl.ds(0, CHUNK)] = lax.shift_right_logical(p, PACK_SHIFT)

    def issue_gab(c, s):
        pltpu.async_copy(a_hbm.at[aidx[s]], abuf[s], sems[s][0])
        pltpu.async_copy(b_hbm.at[didx[s]], bbuf[s], sems[s][1])

    def issue_ge(c, s):
        pltpu.async_copy(ew_hbm.at[pl.ds(ebase + c * CHUNK, CHUNK)],
                         ebuf[s], sems[s][2])

    def wait_g(c, s):
        pltpu.make_async_copy(a_hbm.at[aidx[s]], abuf[s], sems[s][0]).wait()
        pltpu.make_async_copy(b_hbm.at[didx[s]], bbuf[s], sems[s][1]).wait()
        pltpu.make_async_copy(ew_hbm.at[pl.ds(ebase + c * CHUNK, CHUNK)],
                              ebuf[s], sems[s][2]).wait()

    def issue_sc(s):
        sidx[s][pl.ds(0, CHUNK)] = didx[s][pl.ds(0, CHUNK)]
        pltpu.async_copy(ebuf[s], acc.at[sidx[s]], sems[s][3], add=True)

    def wait_sc(s):
        pltpu.make_async_copy(ebuf[s], acc.at[sidx[s]], sems[s][3]).wait()

    def compute(s):
        a_v, b_v, e_v = abuf[s], bbuf[s], ebuf[s]

        @plsc.parallel_loop(0, CHUNK, 1, unroll=8)
        def _row(r):
            for j in range(HID // 16):
                sl = pl.ds(j * 16, 16)
                v = a_v[r, sl] + b_v[r, sl] + e_v[r, sl]
                e_v[r, sl] = jnp.maximum(v, 0.0)

    def process(c, s, issue_next, first=False):
        if issue_next:
            s2 = (s + 2) % NBUF
            unpack(c + 2, s2)
            issue_gab(c + 2, s2)
        wait_g(c, s)
        compute(s)
        issue_sc(s)
        if issue_next:
            if not first:
                wait_sc(s2)
            issue_ge(c + 2, s2)

    unpack(0, 0)
    issue_gab(0, 0)
    issue_ge(0, 0)
    unpack(1, 1)
    issue_gab(1, 1)
    issue_ge(1, 1)
    process(0, 0, True, first=True)
    process(1, 1, True)
    process(2, 2, True)

    def triple(t, carry):
        c0 = t * NBUF
        for k in range(NBUF):
            process(c0 + k, k, True)
        return carry

    lax.fori_loop(1, (NCHUNKS - 4) // NBUF, triple, 0)

    process(jnp.int32(NCHUNKS - 4), 0, True)
    process(jnp.int32(NCHUNKS - 3), 1, True)
    process(jnp.int32(NCHUNKS - 2), 2, False)
    process(jnp.int32(NCHUNKS - 1), 0, False)
    wait_sc(1)
    wait_sc(2)
    wait_sc(0)

    plsc.subcore_barrier()
    pltpu.sync_copy(acc.at[pl.ds(row0, ROW_A)],
                    out_hbm.at[cid, pl.ds(row0, ROW_A)])


def _sc_body_wrapper(a_hbm, b_hbm, ew_hbm, packed_hbm, zeros_hbm, out_hbm,
                     packed_v,
                     ai0, di0, si0, ai1, di1, si1, ai2, di2, si2,
                     a0, b0, e0, a1, b1, e1, a2, b2, e2,
                     sga0, sgb0, sge0, ssc0,
                     sga1, sgb1, sge1, ssc1,
                     sga2, sgb2, sge2, ssc2,
                     acc):
    _sc_layer_body(a_hbm, b_hbm, ew_hbm, packed_hbm, zeros_hbm, out_hbm,
                   packed_v, (ai0, ai1, ai2), (di0, di1, di2),
                   (si0, si1, si2),
                   (a0, a1, a2), (b0, b1, b2), (e0, e1, e2), acc,
                   ((sga0, sgb0, sge0, ssc0),
                    (sga1, sgb1, sge1, ssc1),
                    (sga2, sgb2, sge2, ssc2)))


_sc_layer = pl.kernel(
    _sc_body_wrapper,
    out_type=jax.ShapeDtypeStruct((NCORES, N_PAD, HID), _f32),
    mesh=plsc.VectorSubcoreMesh(core_axis_name="c", subcore_axis_name="s"),
    scratch_types=(
        [pltpu.VMEM((EDGES_PER_TILE,), jnp.int32)]
        + [pltpu.VMEM((CHUNK,), jnp.int32)] * 9
        + [pltpu.VMEM((CHUNK, HID), _f32)] * 9
        + [pltpu.SemaphoreType.DMA] * 12
        + [pltpu.VMEM_SHARED((N_PAD, HID), _f32)]
    ),
)


def kernel(node_coords, edge_index, edge_distances, W_node, b_node, W_edge,
           b_edge, W_msg, b_msg, W_upd, b_upd, ln_scale, ln_bias, W_out, b_out):
    src = edge_index[0].astype(jnp.int32)
    dst = edge_index[1].astype(jnp.int32)
    packed = src | (dst << PACK_SHIFT)
    dists = edge_distances.astype(_f32)

    coords_p = jnp.zeros((N_NODES, HID), _f32).at[:, :3].set(node_coords)
    Wn_p = jnp.zeros((HID, HID), _f32).at[:3].set(W_node)
    W1 = W_msg[:, :HID, :]
    W2 = W_msg[:, HID:2 * HID, :]
    W3 = W_msg[:, 2 * HID:, :]
    Wo_p = jnp.zeros((HID, HID), _f32).at[:, :NV].set(W_out)
    bo_p = jnp.zeros((1, HID), _f32).at[0, :NV].set(b_out)

    eW = _edge_feat(dists, W_edge, b_edge.reshape(1, HID), W3, b_msg)
    h, A, B = _node_embed(coords_p, Wn_p, b_node.reshape(1, HID), W1[0], W2[0])
    zeros_nh = jnp.zeros((N_PAD, HID), _f32)

    logits = None
    for l in range(NLAYERS):
        parts = _sc_layer(A, B, eW[l], packed, zeros_nh)
        if l + 1 < NLAYERS:
            h, A, B = _update_ab(parts, h, W_upd[l], b_upd[l].reshape(1, HID),
                                 ln_scale[l].reshape(1, HID),
                                 ln_bias[l].reshape(1, HID), W1[l + 1], W2[l + 1])
        else:
            logits = _update_logits(parts, h, W_upd[l],
                                    b_upd[l].reshape(1, HID),
                                    ln_scale[l].reshape(1, HID),
                                    ln_bias[l].reshape(1, HID), Wo_p, bo_p)
    return logits[:, :NV]
